# Initial kernel scaffold; baseline (speedup 1.0000x reference)
import functools

import jax
import jax.numpy as jnp
from jax import lax
from jax.experimental import pallas as pl
from jax.experimental.pallas import tpu as pltpu
from jax.experimental.pallas import tpu_sc as plsc

N_USERS = 25000
N_ITEMS = 25000
N = N_USERS + N_ITEMS
H = 32
NUM_TILES = 16
B = 128
SB_ROWS = 8
ROWS_OUT = N // NUM_TILES
CH = 625
NCH = ROWS_OUT // CH
N_LAYERS = 3


def _sc_body(e0b, src2, dst2, w2, zeros, e1b, e2b, e3b,
             sbuf_src, sbuf_dst, sbuf_w, rows, zbuf, vbuf, acc, sem):
    c = lax.axis_index("c")
    tid = lax.axis_index("s")
    half_base = c * N
    out_base = tid * ROWS_OUT
    rows_per_tile_e = src2.shape[0] // NUM_TILES
    n_super = rows_per_tile_e // SB_ROWS

    pltpu.sync_copy(zeros, zbuf)

    gather_tabs = [e0b, e1b, e2b]
    out_tabs = [e1b, e2b, e3b]
    for l in range(N_LAYERS):
        for ch in range(NCH):
            pltpu.sync_copy(zbuf, acc.at[pl.ds(out_base + ch * CH, CH)])
        plsc.subcore_barrier()

        src_tab = gather_tabs[l]

        def k_body(k, _):
            for m in range(8):
                sl = pl.ds(m * 16, 16)
                sbuf_src[k, sl] = sbuf_src[k, sl] + half_base
            pltpu.async_copy(src_tab.at[sbuf_src.at[k]], rows, sem).wait()
            for m in range(8):
                w16 = sbuf_w[k, pl.ds(m * 16, 16)]
                for j in range(16):
                    e = m * 16 + j
                    wv = w16[j]
                    rows[e, pl.ds(0, 16)] = rows[e, pl.ds(0, 16)] * wv
                    rows[e, pl.ds(16, 16)] = rows[e, pl.ds(16, 16)] * wv
            pltpu.sync_copy(rows, acc.at[sbuf_dst.at[k]], add=True)
            return 0

        def sb_body(g, _):
            row0 = tid * rows_per_tile_e + g * SB_ROWS
            pltpu.sync_copy(src2.at[pl.ds(row0, SB_ROWS)], sbuf_src)
            pltpu.sync_copy(dst2.at[pl.ds(row0, SB_ROWS)], sbuf_dst)
            pltpu.sync_copy(w2.at[pl.ds(row0, SB_ROWS)], sbuf_w)
            lax.fori_loop(0, SB_ROWS, k_body, 0)
            return 0

        lax.fori_loop(0, n_super, sb_body, 0)
        plsc.subcore_barrier()

        dst_tab = out_tabs[l]
        for ch in range(NCH):
            r0 = out_base + ch * CH
            pltpu.sync_copy(acc.at[pl.ds(r0, CH)], vbuf)
            pltpu.sync_copy(vbuf, dst_tab.at[pl.ds(half_base + r0, CH)])
        plsc.subcore_barrier()


def _mean_body(a_ref, b_ref, c_ref, d_ref, o_ref):
    o_ref[...] = (a_ref[...] + b_ref[...] + c_ref[...] + d_ref[...]) * 0.25


def kernel(edge_index, edge_weight, user_emb, item_emb):
    dst = edge_index[0].astype(jnp.int32)
    src = edge_index[1].astype(jnp.int32)
    w = edge_weight.astype(jnp.float32)

    n_edges = w.shape[0]
    per_tile = NUM_TILES * B * SB_ROWS
    e_pad = ((n_edges + per_tile - 1) // per_tile) * per_tile
    pad = e_pad - n_edges
    if pad:
        src = jnp.concatenate([src, jnp.zeros((pad,), jnp.int32)])
        dst = jnp.concatenate([dst, jnp.zeros((pad,), jnp.int32)])
        w = jnp.concatenate([w, jnp.zeros((pad,), jnp.float32)])
    src2 = src.reshape(-1, B)
    dst2 = dst.reshape(-1, B)
    w2 = w.reshape(-1, B)

    ego = jnp.concatenate([user_emb, item_emb], axis=0)
    e0b = jnp.concatenate([ego[:, :H], ego[:, H:]], axis=0)
    zeros = jnp.zeros((CH, H), jnp.float32)

    mesh = plsc.VectorSubcoreMesh(core_axis_name="c", subcore_axis_name="s")
    out_t = [jax.ShapeDtypeStruct((2 * N, H), jnp.float32)] * 3
    e1b, e2b, e3b = pl.kernel(
        _sc_body,
        out_type=out_t,
        mesh=mesh,
        scratch_types=[
            pltpu.VMEM((SB_ROWS, B), jnp.int32),
            pltpu.VMEM((SB_ROWS, B), jnp.int32),
            pltpu.VMEM((SB_ROWS, B), jnp.float32),
            pltpu.VMEM((B, H), jnp.float32),
            pltpu.VMEM((CH, H), jnp.float32),
            pltpu.VMEM((CH, H), jnp.float32),
            pltpu.VMEM_SHARED((N, H), jnp.float32),
            pltpu.SemaphoreType.DMA,
        ],
    )(e0b, src2, dst2, w2, zeros)

    blk = 4000
    grid = (2 * N) // blk
    spec = pl.BlockSpec((blk, H), lambda i: (i, 0))
    mean_b = pl.pallas_call(
        _mean_body,
        grid=(grid,),
        in_specs=[spec] * 4,
        out_specs=spec,
        out_shape=jax.ShapeDtypeStruct((2 * N, H), jnp.float32),
    )(e0b, e1b, e2b, e3b)

    mean = jnp.concatenate([mean_b[:N], mean_b[N:]], axis=1)
    return mean[:N_USERS], mean[N_USERS:]


# speedup vs baseline: 5.2651x; 5.2651x over previous
import functools

import jax
import jax.numpy as jnp
from jax import lax
from jax.experimental import pallas as pl
from jax.experimental.pallas import tpu as pltpu
from jax.experimental.pallas import tpu_sc as plsc

N_USERS = 25000
N_ITEMS = 25000
N = N_USERS + N_ITEMS
H = 32
NUM_TILES = 16
B = 128
SB_ROWS = 8
ROWS_OUT = 3136
N_PAD = NUM_TILES * ROWS_OUT
CH = 224
NCH = ROWS_OUT // CH
N_LAYERS = 3


def _sc_body(e0b, src2, dst2, w2, zeros, e1b, e2b, e3b,
             sbuf_src, sbuf_dst, sbuf_w, rows, vbuf, acc, sem):
    c = lax.axis_index("c")
    tid = lax.axis_index("s")
    half_base = c * N_PAD
    out_base = tid * ROWS_OUT
    rows_per_tile_e = src2.shape[0] // NUM_TILES
    n_super = rows_per_tile_e // SB_ROWS

    gather_tabs = [e0b, e1b, e2b]
    out_tabs = [e1b, e2b, e3b]
    for l in range(N_LAYERS):
        pltpu.sync_copy(zeros, vbuf)
        for ch in range(NCH):
            pltpu.sync_copy(vbuf, acc.at[pl.ds(out_base + ch * CH, CH)])
        plsc.subcore_barrier()

        src_tab = gather_tabs[l]

        def k_body(k, _):
            for m in range(8):
                sl = pl.ds(m * 16, 16)
                sbuf_src[k, sl] = sbuf_src[k, sl] + half_base
            pltpu.async_copy(src_tab.at[sbuf_src.at[k]], rows, sem).wait()
            for m in range(8):
                w16 = sbuf_w[k, pl.ds(m * 16, 16)]
                for j in range(16):
                    e = m * 16 + j
                    wv = w16[j]
                    rows[e, pl.ds(0, 16)] = rows[e, pl.ds(0, 16)] * wv
                    rows[e, pl.ds(16, 16)] = rows[e, pl.ds(16, 16)] * wv
            pltpu.sync_copy(rows, acc.at[sbuf_dst.at[k]], add=True)
            return 0

        def sb_body(g, _):
            row0 = tid * rows_per_tile_e + g * SB_ROWS
            pltpu.sync_copy(src2.at[pl.ds(row0, SB_ROWS)], sbuf_src)
            pltpu.sync_copy(dst2.at[pl.ds(row0, SB_ROWS)], sbuf_dst)
            pltpu.sync_copy(w2.at[pl.ds(row0, SB_ROWS)], sbuf_w)
            lax.fori_loop(0, SB_ROWS, k_body, 0)
            return 0

        lax.fori_loop(0, n_super, sb_body, 0)
        plsc.subcore_barrier()

        dst_tab = out_tabs[l]
        for ch in range(NCH):
            r0 = out_base + ch * CH
            pltpu.sync_copy(acc.at[pl.ds(r0, CH)], vbuf)
            pltpu.sync_copy(vbuf, dst_tab.at[pl.ds(half_base + r0, CH)])
        plsc.subcore_barrier()


def _mean_body(a_ref, b_ref, c_ref, d_ref, o_ref):
    o_ref[...] = (a_ref[...] + b_ref[...] + c_ref[...] + d_ref[...]) * 0.25


def kernel(edge_index, edge_weight, user_emb, item_emb):
    dst = edge_index[0].astype(jnp.int32)
    src = edge_index[1].astype(jnp.int32)
    w = edge_weight.astype(jnp.float32)

    n_edges = w.shape[0]
    per_tile = NUM_TILES * B * SB_ROWS
    e_pad = ((n_edges + per_tile - 1) // per_tile) * per_tile
    pad = e_pad - n_edges
    if pad:
        src = jnp.concatenate([src, jnp.zeros((pad,), jnp.int32)])
        dst = jnp.concatenate([dst, jnp.zeros((pad,), jnp.int32)])
        w = jnp.concatenate([w, jnp.zeros((pad,), jnp.float32)])
    src2 = src.reshape(-1, B)
    dst2 = dst.reshape(-1, B)
    w2 = w.reshape(-1, B)

    ego = jnp.concatenate(
        [user_emb, item_emb, jnp.zeros((N_PAD - N, 2 * H), jnp.float32)],
        axis=0)
    e0b = jnp.concatenate([ego[:, :H], ego[:, H:]], axis=0)
    zeros = jnp.zeros((CH, H), jnp.float32)

    mesh = plsc.VectorSubcoreMesh(core_axis_name="c", subcore_axis_name="s")
    out_t = [jax.ShapeDtypeStruct((2 * N_PAD, H), jnp.float32)] * 3
    e1b, e2b, e3b = pl.kernel(
        _sc_body,
        out_type=out_t,
        mesh=mesh,
        compiler_params=pltpu.CompilerParams(use_tc_tiling_on_sc=False),
        scratch_types=[
            pltpu.VMEM((SB_ROWS, B), jnp.int32),
            pltpu.VMEM((SB_ROWS, B), jnp.int32),
            pltpu.VMEM((SB_ROWS, B), jnp.float32),
            pltpu.VMEM((B, H), jnp.float32),
            pltpu.VMEM((CH, H), jnp.float32),
            pltpu.VMEM_SHARED((N_PAD, H), jnp.float32),
            pltpu.SemaphoreType.DMA,
        ],
    )(e0b, src2, dst2, w2, zeros)

    blk = 784
    grid = (2 * N_PAD) // blk
    spec = pl.BlockSpec((blk, H), lambda i: (i, 0))
    mean_b = pl.pallas_call(
        _mean_body,
        grid=(grid,),
        in_specs=[spec] * 4,
        out_specs=spec,
        out_shape=jax.ShapeDtypeStruct((2 * N_PAD, H), jnp.float32),
    )(e0b, e1b, e2b, e3b)

    mean = jnp.concatenate(
        [mean_b[:N], mean_b[N_PAD:N_PAD + N]], axis=1)
    return mean[:N_USERS], mean[N_USERS:]


# speedup vs baseline: 9.7274x; 1.8475x over previous
import functools

import jax
import jax.numpy as jnp
from jax import lax
from jax.experimental import pallas as pl
from jax.experimental.pallas import tpu as pltpu
from jax.experimental.pallas import tpu_sc as plsc

N_USERS = 25000
N_ITEMS = 25000
N = N_USERS + N_ITEMS
H = 32
NUM_TILES = 16
B = 128
SB_ROWS = 8
ROWS_OUT = 3136
N_PAD = NUM_TILES * ROWS_OUT
CH = 112
NCH = ROWS_OUT // CH
N_LAYERS = 3


def _sc_body(e0b, src2c, dst2, w2, zeros, e1b, e2b, e3b,
             src_b, dst_b, w_b, rows3, wb0, wb1, acc,
             sem_g, sem_s, sem_i, sem_r, sem_w):
    c = lax.axis_index("c")
    tid = lax.axis_index("s")
    half_base = c * N_PAD
    out_base = tid * ROWS_OUT
    rows_e = dst2.shape[0] // NUM_TILES
    n_sb = rows_e // SB_ROWS

    def acc_body(acc):
        def idx_copies(sp, gb):
            r0s = c * dst2.shape[0] + tid * rows_e + sp * SB_ROWS
            r0d = tid * rows_e + sp * SB_ROWS
            return [
                pltpu.make_async_copy(
                    src2c.at[pl.ds(r0s, SB_ROWS)], src_b.at[gb], sem_i.at[gb]),
                pltpu.make_async_copy(
                    dst2.at[pl.ds(r0d, SB_ROWS)], dst_b.at[gb], sem_i.at[gb]),
                pltpu.make_async_copy(
                    w2.at[pl.ds(r0d, SB_ROWS)], w_b.at[gb], sem_i.at[gb]),
            ]

        gather_tabs = [e0b, e1b, e2b]
        out_tabs = [e1b, e2b, e3b]
        for l in range(N_LAYERS):
            pltpu.sync_copy(zeros, wb0)
            zdescs = [
                pltpu.make_async_copy(
                    wb0, acc.at[pl.ds(out_base + ch * CH, CH)], sem_r)
                for ch in range(NCH)
            ]
            for d in zdescs:
                d.start()
            for d in zdescs:
                d.wait()
            plsc.subcore_barrier()

            src_tab = gather_tabs[l]

            def gather_desc(rr_, gb_, b_):
                return pltpu.make_async_copy(
                    src_tab.at[src_b.at[gb_, rr_]], rows3.at[b_], sem_g.at[b_])

            def scatter_desc(rr_, gb_, b_):
                return pltpu.make_async_copy(
                    rows3.at[b_], acc.at[dst_b.at[gb_, rr_]], sem_s.at[b_])

            for d in idx_copies(0, 0):
                d.start()
            for d in idx_copies(0, 0):
                d.wait()
            for d in idx_copies(1, 1):
                d.start()
            gather_desc(0, 0, 0).start()
            gather_desc(1, 0, 1).start()

            def r_body(r, _):
                s = lax.shift_right_logical(r, 3)
                rr = lax.rem(r, SB_ROWS)
                gb = lax.rem(s, 2)
                b = lax.rem(r, 3)
                gather_desc(rr, gb, b).wait()
                rv = rows3.at[b]
                wv_row = w_b.at[gb, rr]
                for m in range(8):
                    w16 = wv_row[pl.ds(m * 16, 16)]
                    for j in range(16):
                        e = m * 16 + j
                        wj = w16[j]
                        rv[e, pl.ds(0, 16)] = rv[e, pl.ds(0, 16)] * wj
                        rv[e, pl.ds(16, 16)] = rv[e, pl.ds(16, 16)] * wj
                pltpu.async_copy(
                    rows3.at[b], acc.at[dst_b.at[gb, rr]], sem_s.at[b],
                    add=True)

                @pl.when((rr == 3) & (s >= 1) & (s < n_sb - 1))
                def _():
                    sp = s + 1
                    for d in idx_copies(sp, lax.rem(sp, 2)):
                        d.start()

                @pl.when(r + 2 < rows_e)
                def _():
                    r2 = r + 2
                    s2 = lax.shift_right_logical(r2, 3)
                    rr2 = lax.rem(r2, SB_ROWS)
                    gb2 = lax.rem(s2, 2)
                    b2 = lax.rem(r2, 3)

                    @pl.when(r >= 1)
                    def _():
                        scatter_desc(rr2, gb2, b2).wait()

                    @pl.when(rr2 == 0)
                    def _():
                        for d in idx_copies(s2, gb2):
                            d.wait()

                    gather_desc(rr2, gb2, b2).start()

                return 0

            lax.fori_loop(0, rows_e, r_body, 0)
            for back in (3, 2, 1):
                scatter_desc(SB_ROWS - back, lax.rem(n_sb - 1, 2),
                             lax.rem(rows_e - back, 3)).wait()
            plsc.subcore_barrier()

            dst_tab = out_tabs[l]

            def rd(ch, buf):
                return pltpu.make_async_copy(
                    acc.at[pl.ds(out_base + ch * CH, CH)], buf, sem_r)

            def wr(ch, buf):
                return pltpu.make_async_copy(
                    buf, dst_tab.at[pl.ds(half_base + out_base + ch * CH, CH)],
                    sem_w)

            bufs = [wb0, wb1]
            rd(0, wb0).start()
            for ch in range(NCH):
                p = bufs[ch % 2]
                q = bufs[(ch + 1) % 2]
                rd(ch, p).wait()
                if ch >= 1:
                    wr(ch - 1, q).wait()
                if ch + 1 < NCH:
                    rd(ch + 1, q).start()
                wr(ch, p).start()
            wr(NCH - 1, bufs[(NCH - 1) % 2]).wait()
            plsc.subcore_barrier()

    acc_body(acc)


def _mean_body(a_ref, b_ref, c_ref, d_ref, o_ref):
    o_ref[...] = (a_ref[...] + b_ref[...] + c_ref[...] + d_ref[...]) * 0.25


def kernel(edge_index, edge_weight, user_emb, item_emb):
    dst = edge_index[0].astype(jnp.int32)
    src = edge_index[1].astype(jnp.int32)
    w = edge_weight.astype(jnp.float32)

    n_edges = w.shape[0]
    per_tile = NUM_TILES * B * SB_ROWS
    e_pad = ((n_edges + per_tile - 1) // per_tile) * per_tile
    pad = e_pad - n_edges
    if pad:
        src = jnp.concatenate([src, jnp.zeros((pad,), jnp.int32)])
        dst = jnp.concatenate([dst, jnp.zeros((pad,), jnp.int32)])
        w = jnp.concatenate([w, jnp.zeros((pad,), jnp.float32)])
    src2 = src.reshape(-1, B)
    dst2 = dst.reshape(-1, B)
    w2 = w.reshape(-1, B)

    ego = jnp.concatenate(
        [user_emb, item_emb, jnp.zeros((N_PAD - N, 2 * H), jnp.float32)],
        axis=0)
    e0b = jnp.concatenate([ego[:, :H], ego[:, H:]], axis=0)
    zeros = jnp.zeros((CH, H), jnp.float32)

    src2c = jnp.concatenate([src2, src2 + N_PAD], axis=0)

    mesh = plsc.VectorSubcoreMesh(core_axis_name="c", subcore_axis_name="s")
    out_t = [jax.ShapeDtypeStruct((2 * N_PAD, H), jnp.float32)] * 3
    e1b, e2b, e3b = pl.kernel(
        _sc_body,
        out_type=out_t,
        mesh=mesh,
        compiler_params=pltpu.CompilerParams(use_tc_tiling_on_sc=False),
        scratch_types=[
            pltpu.VMEM((2, SB_ROWS, B), jnp.int32),
            pltpu.VMEM((2, SB_ROWS, B), jnp.int32),
            pltpu.VMEM((2, SB_ROWS, B), jnp.float32),
            pltpu.VMEM((3, B, H), jnp.float32),
            pltpu.VMEM((CH, H), jnp.float32),
            pltpu.VMEM((CH, H), jnp.float32),
            pltpu.VMEM_SHARED((N_PAD, H), jnp.float32),
            pltpu.SemaphoreType.DMA((3,)),
            pltpu.SemaphoreType.DMA((3,)),
            pltpu.SemaphoreType.DMA((2,)),
            pltpu.SemaphoreType.DMA,
            pltpu.SemaphoreType.DMA,
        ],
    )(e0b, src2c, dst2, w2, zeros)

    blk = 784
    grid = (2 * N_PAD) // blk
    spec = pl.BlockSpec((blk, H), lambda i: (i, 0))
    mean_b = pl.pallas_call(
        _mean_body,
        grid=(grid,),
        in_specs=[spec] * 4,
        out_specs=spec,
        out_shape=jax.ShapeDtypeStruct((2 * N_PAD, H), jnp.float32),
    )(e0b, e1b, e2b, e3b)

    mean = jnp.concatenate(
        [mean_b[:N], mean_b[N_PAD:N_PAD + N]], axis=1)
    return mean[:N_USERS], mean[N_USERS:]


# speedup vs baseline: 11.1791x; 1.1492x over previous
import functools

import jax
import jax.numpy as jnp
from jax import lax
from jax.experimental import pallas as pl
from jax.experimental.pallas import tpu as pltpu
from jax.experimental.pallas import tpu_sc as plsc

N_USERS = 25000
N_ITEMS = 25000
N = N_USERS + N_ITEMS
H = 32
NUM_TILES = 16
B = 128
SB_ROWS = 8
ROWS_OUT = 3136
N_PAD = NUM_TILES * ROWS_OUT
CH = 56
NCH = ROWS_OUT // CH
N_LAYERS = 3


def _sc_body(e0b, ei3, w2, zeros, e1b, e2b, mout,
             src_b, dst_b, w_b, rows3, wb0, wb1, mb1, mb2, mb3, acc,
             sem_g, sem_s, sem_i, sem_r, sem_w):
    c = lax.axis_index("c")
    tid = lax.axis_index("s")
    half_base = c * N_PAD
    out_base = tid * ROWS_OUT
    rows_e = ei3.shape[1] // NUM_TILES
    n_sb = rows_e // SB_ROWS

    def acc_body(acc):
        def idx_copies(sp, gb):
            r0 = tid * rows_e + sp * SB_ROWS
            return [
                pltpu.make_async_copy(
                    ei3.at[1, pl.ds(r0, SB_ROWS)], src_b.at[gb], sem_i.at[gb]),
                pltpu.make_async_copy(
                    ei3.at[0, pl.ds(r0, SB_ROWS)], dst_b.at[gb], sem_i.at[gb]),
                pltpu.make_async_copy(
                    w2.at[pl.ds(r0, SB_ROWS)], w_b.at[gb], sem_i.at[gb]),
            ]

        def shift_src(gb_, rr_):
            srow = src_b.at[gb_, rr_]
            for m in range(8):
                sl = pl.ds(m * 16, 16)
                srow[sl] = srow[sl] + half_base

        gather_tabs = [e0b, e1b, e2b]
        out_tabs = [e1b, e2b, None]
        for l in range(N_LAYERS):
            pltpu.sync_copy(zeros, wb0)
            zdescs = [
                pltpu.make_async_copy(
                    wb0, acc.at[pl.ds(out_base + ch * CH, CH)], sem_r)
                for ch in range(NCH)
            ]
            for half in (zdescs[:NCH // 2], zdescs[NCH // 2:]):
                for d in half:
                    d.start()
                for d in half:
                    d.wait()
            plsc.subcore_barrier()

            src_tab = gather_tabs[l]

            def gather_desc(rr_, gb_, b_):
                return pltpu.make_async_copy(
                    src_tab.at[src_b.at[gb_, rr_]], rows3.at[b_], sem_g.at[b_])

            def scatter_desc(rr_, gb_, b_):
                return pltpu.make_async_copy(
                    rows3.at[b_], acc.at[dst_b.at[gb_, rr_]], sem_s.at[b_])

            for d in idx_copies(0, 0):
                d.start()
            for d in idx_copies(0, 0):
                d.wait()
            for d in idx_copies(1, 1):
                d.start()
            shift_src(0, 0)
            shift_src(0, 1)
            gather_desc(0, 0, 0).start()
            gather_desc(1, 0, 1).start()

            def r_body(r, _):
                s = lax.shift_right_logical(r, 3)
                rr = lax.rem(r, SB_ROWS)
                gb = lax.rem(s, 2)
                b = lax.rem(r, 3)
                gather_desc(rr, gb, b).wait()
                rv = rows3.at[b]
                wv_row = w_b.at[gb, rr]
                for m in range(8):
                    w16 = wv_row[pl.ds(m * 16, 16)]
                    for j in range(16):
                        e = m * 16 + j
                        wj = w16[j]
                        rv[e, pl.ds(0, 16)] = rv[e, pl.ds(0, 16)] * wj
                        rv[e, pl.ds(16, 16)] = rv[e, pl.ds(16, 16)] * wj
                pltpu.async_copy(
                    rows3.at[b], acc.at[dst_b.at[gb, rr]], sem_s.at[b],
                    add=True)

                @pl.when((rr == 3) & (s >= 1) & (s < n_sb - 1))
                def _():
                    sp = s + 1
                    for d in idx_copies(sp, lax.rem(sp, 2)):
                        d.start()

                @pl.when(r + 2 < rows_e)
                def _():
                    r2 = r + 2
                    s2 = lax.shift_right_logical(r2, 3)
                    rr2 = lax.rem(r2, SB_ROWS)
                    gb2 = lax.rem(s2, 2)
                    b2 = lax.rem(r2, 3)

                    @pl.when(r >= 1)
                    def _():
                        scatter_desc(rr2, gb2, b2).wait()

                    @pl.when(rr2 == 0)
                    def _():
                        for d in idx_copies(s2, gb2):
                            d.wait()

                    shift_src(gb2, rr2)
                    gather_desc(rr2, gb2, b2).start()

                return 0

            lax.fori_loop(0, rows_e, r_body, 0)
            for back in (3, 2, 1):
                scatter_desc(SB_ROWS - back, lax.rem(n_sb - 1, 2),
                             lax.rem(rows_e - back, 3)).wait()
            plsc.subcore_barrier()

            if l < N_LAYERS - 1:
                dst_tab = out_tabs[l]

                def rd(ch, buf):
                    return pltpu.make_async_copy(
                        acc.at[pl.ds(out_base + ch * CH, CH)], buf, sem_r)

                def wr(ch, buf):
                    return pltpu.make_async_copy(
                        buf,
                        dst_tab.at[pl.ds(half_base + out_base + ch * CH, CH)],
                        sem_w)

                bufs = [wb0, wb1]
                rd(0, wb0).start()
                for ch in range(NCH):
                    p = bufs[ch % 2]
                    q = bufs[(ch + 1) % 2]
                    rd(ch, p).wait()
                    if ch >= 1:
                        wr(ch - 1, q).wait()
                    if ch + 1 < NCH:
                        rd(ch + 1, q).start()
                    wr(ch, p).start()
                wr(NCH - 1, bufs[(NCH - 1) % 2]).wait()
                plsc.subcore_barrier()
            else:
                bA = [wb0, mb3]

                def mrds(ch, buf):
                    r0 = out_base + ch * CH
                    hr0 = half_base + r0
                    return [
                        pltpu.make_async_copy(
                            acc.at[pl.ds(r0, CH)], buf, sem_r),
                        pltpu.make_async_copy(
                            e0b.at[pl.ds(hr0, CH)], wb1, sem_g.at[0]),
                        pltpu.make_async_copy(
                            e1b.at[pl.ds(hr0, CH)], mb1, sem_g.at[1]),
                        pltpu.make_async_copy(
                            e2b.at[pl.ds(hr0, CH)], mb2, sem_g.at[2]),
                    ]

                def mwr(ch, buf):
                    return pltpu.make_async_copy(
                        buf,
                        mout.at[pl.ds(half_base + out_base + ch * CH, CH)],
                        sem_w)

                def do_chunk(ch, buf, first):
                    if not first:
                        mwr(ch - 2, buf).wait()
                    for d in mrds(ch, buf):
                        d.start()
                    for d in mrds(ch, buf):
                        d.wait()

                    def mrow(i, _):
                        for off in (0, 16):
                            sl = pl.ds(off, 16)
                            buf[i, sl] = (buf[i, sl] + wb1[i, sl]
                                          + mb1[i, sl] + mb2[i, sl]) * 0.25
                        return 0

                    lax.fori_loop(0, CH, mrow, 0)
                    mwr(ch, buf).start()

                for ch in range(NCH):
                    do_chunk(ch, bA[ch % 2], ch < 2)
                mwr(NCH - 2, bA[0]).wait()
                mwr(NCH - 1, bA[1]).wait()
                plsc.subcore_barrier()

    acc_body(acc)


def _mean_body(a_ref, b_ref, c_ref, d_ref, o_ref):
    o_ref[...] = (a_ref[...] + b_ref[...] + c_ref[...] + d_ref[...]) * 0.25


def kernel(edge_index, edge_weight, user_emb, item_emb):
    ei = edge_index.astype(jnp.int32)
    w = edge_weight.astype(jnp.float32)

    n_edges = w.shape[0]
    per_tile = NUM_TILES * B * SB_ROWS
    e_pad = ((n_edges + per_tile - 1) // per_tile) * per_tile
    pad = e_pad - n_edges
    if pad:
        ei = jnp.concatenate([ei, jnp.zeros((2, pad), jnp.int32)], axis=1)
        w = jnp.concatenate([w, jnp.zeros((pad,), jnp.float32)])
    ei3 = ei.reshape(2, -1, B)
    w2 = w.reshape(-1, B)

    ego = jnp.concatenate(
        [user_emb, item_emb, jnp.zeros((N_PAD - N, 2 * H), jnp.float32)],
        axis=0)
    e0b = jnp.concatenate([ego[:, :H], ego[:, H:]], axis=0)
    zeros = jnp.zeros((CH, H), jnp.float32)

    mesh = plsc.VectorSubcoreMesh(core_axis_name="c", subcore_axis_name="s")
    out_t = [jax.ShapeDtypeStruct((2 * N_PAD, H), jnp.float32)] * 3
    _, _, mout = pl.kernel(
        _sc_body,
        out_type=out_t,
        mesh=mesh,
        compiler_params=pltpu.CompilerParams(use_tc_tiling_on_sc=False),
        scratch_types=[
            pltpu.VMEM((2, SB_ROWS, B), jnp.int32),
            pltpu.VMEM((2, SB_ROWS, B), jnp.int32),
            pltpu.VMEM((2, SB_ROWS, B), jnp.float32),
            pltpu.VMEM((3, B, H), jnp.float32),
            pltpu.VMEM((CH, H), jnp.float32),
            pltpu.VMEM((CH, H), jnp.float32),
            pltpu.VMEM((CH, H), jnp.float32),
            pltpu.VMEM((CH, H), jnp.float32),
            pltpu.VMEM((CH, H), jnp.float32),
            pltpu.VMEM_SHARED((N_PAD, H), jnp.float32),
            pltpu.SemaphoreType.DMA((3,)),
            pltpu.SemaphoreType.DMA((3,)),
            pltpu.SemaphoreType.DMA((2,)),
            pltpu.SemaphoreType.DMA,
            pltpu.SemaphoreType.DMA,
        ],
    )(e0b, ei3, w2, zeros)

    mean = jnp.concatenate(
        [mout[:N], mout[N_PAD:N_PAD + N]], axis=1)
    return mean[:N_USERS], mean[N_USERS:]


# speedup vs baseline: 11.7627x; 1.0522x over previous
import functools

import jax
import jax.numpy as jnp
from jax import lax
from jax.experimental import pallas as pl
from jax.experimental.pallas import tpu as pltpu
from jax.experimental.pallas import tpu_sc as plsc

N_USERS = 25000
N_ITEMS = 25000
N = N_USERS + N_ITEMS
H = 32
NUM_TILES = 16
B = 128
SB_ROWS = 8
ROWS_OUT = 3136
N_PAD = NUM_TILES * ROWS_OUT
CH = 56
NCH = ROWS_OUT // CH
N_LAYERS = 3


def _sc_body(e0b, ei3, w2, zeros, e1b, e2b, mout,
             src_b, dst_b, w_b, rows3, wb0, wb1, mb1, mb2, mb3, acc,
             sem_g, sem_s, sem_i, sem_r, sem_w):
    c = lax.axis_index("c")
    tid = lax.axis_index("s")
    half_base = c * N_PAD
    out_base = tid * ROWS_OUT
    rows_e = ei3.shape[1] // NUM_TILES
    n_sb = rows_e // SB_ROWS

    def acc_body(acc):
        def idx_copies(sp, gb):
            r0 = tid * rows_e + sp * SB_ROWS
            return [
                pltpu.make_async_copy(
                    ei3.at[1, pl.ds(r0, SB_ROWS)], src_b.at[gb], sem_i.at[gb]),
                pltpu.make_async_copy(
                    ei3.at[0, pl.ds(r0, SB_ROWS)], dst_b.at[gb], sem_i.at[gb]),
                pltpu.make_async_copy(
                    w2.at[pl.ds(r0, SB_ROWS)], w_b.at[gb], sem_i.at[gb]),
            ]

        def shift_src(gb_, rr_):
            srow = src_b.at[gb_, rr_]
            for m in range(8):
                sl = pl.ds(m * 16, 16)
                srow[sl] = srow[sl] + half_base

        gather_tabs = [e0b, e1b, e2b]
        out_tabs = [e1b, e2b, None]
        for l in range(N_LAYERS):
            pltpu.sync_copy(zeros, wb0)
            zdescs = [
                pltpu.make_async_copy(
                    wb0, acc.at[pl.ds(out_base + ch * CH, CH)], sem_r)
                for ch in range(NCH)
            ]
            for half in (zdescs[:NCH // 2], zdescs[NCH // 2:]):
                for d in half:
                    d.start()
                for d in half:
                    d.wait()
            plsc.subcore_barrier()

            src_tab = gather_tabs[l]

            def gather_desc(rr_, gb_, b_):
                return pltpu.make_async_copy(
                    src_tab.at[src_b.at[gb_, rr_]], rows3.at[b_], sem_g.at[b_])

            def scatter_desc(rr_, gb_, b_):
                return pltpu.make_async_copy(
                    rows3.at[b_], acc.at[dst_b.at[gb_, rr_]], sem_s.at[b_])

            for d in idx_copies(0, 0):
                d.start()
            for d in idx_copies(0, 0):
                d.wait()
            for d in idx_copies(1, 1):
                d.start()
            shift_src(0, 0)
            shift_src(0, 1)
            gather_desc(0, 0, 0).start()
            gather_desc(1, 0, 1).start()

            def r_body(r, _):
                s = lax.shift_right_logical(r, 3)
                rr = lax.rem(r, SB_ROWS)
                gb = lax.rem(s, 2)
                b = lax.rem(r, 3)
                gather_desc(rr, gb, b).wait()
                rv = rows3.at[b]
                wv_row = w_b.at[gb, rr]
                for m in range(8):
                    w16 = wv_row[pl.ds(m * 16, 16)]
                    for j in range(16):
                        e = m * 16 + j
                        wj = w16[j]
                        rv[e, pl.ds(0, 16)] = rv[e, pl.ds(0, 16)] * wj
                        rv[e, pl.ds(16, 16)] = rv[e, pl.ds(16, 16)] * wj
                pltpu.async_copy(
                    rows3.at[b], acc.at[dst_b.at[gb, rr]], sem_s.at[b],
                    add=True)

                @pl.when((rr == 3) & (s >= 1) & (s < n_sb - 1))
                def _():
                    sp = s + 1
                    for d in idx_copies(sp, lax.rem(sp, 2)):
                        d.start()

                @pl.when(r + 2 < rows_e)
                def _():
                    r2 = r + 2
                    s2 = lax.shift_right_logical(r2, 3)
                    rr2 = lax.rem(r2, SB_ROWS)
                    gb2 = lax.rem(s2, 2)
                    b2 = lax.rem(r2, 3)

                    @pl.when(r >= 1)
                    def _():
                        scatter_desc(rr2, gb2, b2).wait()

                    @pl.when(rr2 == 0)
                    def _():
                        for d in idx_copies(s2, gb2):
                            d.wait()

                    shift_src(gb2, rr2)
                    gather_desc(rr2, gb2, b2).start()

                return 0

            lax.fori_loop(0, rows_e, r_body, 0)
            for back in (3, 2, 1):
                scatter_desc(SB_ROWS - back, lax.rem(n_sb - 1, 2),
                             lax.rem(rows_e - back, 3)).wait()
            plsc.subcore_barrier()

            if l < N_LAYERS - 1:
                dst_tab = out_tabs[l]

                def rd(ch, buf):
                    return pltpu.make_async_copy(
                        acc.at[pl.ds(out_base + ch * CH, CH)], buf, sem_r)

                def wr(ch, buf):
                    return pltpu.make_async_copy(
                        buf,
                        dst_tab.at[pl.ds(half_base + out_base + ch * CH, CH)],
                        sem_w)

                bufs = [wb0, wb1]
                rd(0, wb0).start()
                for ch in range(NCH):
                    p = bufs[ch % 2]
                    q = bufs[(ch + 1) % 2]
                    rd(ch, p).wait()
                    if ch >= 1:
                        wr(ch - 1, q).wait()
                    if ch + 1 < NCH:
                        rd(ch + 1, q).start()
                    wr(ch, p).start()
                wr(NCH - 1, bufs[(NCH - 1) % 2]).wait()
                plsc.subcore_barrier()
            else:
                bA = [wb0, mb3]

                def mrds(ch, buf):
                    r0 = out_base + ch * CH
                    hr0 = half_base + r0
                    return [
                        pltpu.make_async_copy(
                            acc.at[pl.ds(r0, CH)], buf, sem_r),
                        pltpu.make_async_copy(
                            e0b.at[pl.ds(hr0, CH)], wb1, sem_g.at[0]),
                        pltpu.make_async_copy(
                            e1b.at[pl.ds(hr0, CH)], mb1, sem_g.at[1]),
                        pltpu.make_async_copy(
                            e2b.at[pl.ds(hr0, CH)], mb2, sem_g.at[2]),
                    ]

                def mwr(ch, buf):
                    return pltpu.make_async_copy(
                        buf,
                        mout.at[pl.ds(out_base + ch * CH, CH),
                                pl.ds(c * H, H)],
                        sem_w)

                def do_chunk(ch, buf, first):
                    if not first:
                        mwr(ch - 2, buf).wait()
                    for d in mrds(ch, buf):
                        d.start()
                    for d in mrds(ch, buf):
                        d.wait()

                    def mrow(i, _):
                        for off in (0, 16):
                            sl = pl.ds(off, 16)
                            buf[i, sl] = (buf[i, sl] + wb1[i, sl]
                                          + mb1[i, sl] + mb2[i, sl]) * 0.25
                        return 0

                    lax.fori_loop(0, CH, mrow, 0)
                    mwr(ch, buf).start()

                for ch in range(NCH):
                    do_chunk(ch, bA[ch % 2], ch < 2)
                mwr(NCH - 2, bA[0]).wait()
                mwr(NCH - 1, bA[1]).wait()
                plsc.subcore_barrier()

    acc_body(acc)


def _mean_body(a_ref, b_ref, c_ref, d_ref, o_ref):
    o_ref[...] = (a_ref[...] + b_ref[...] + c_ref[...] + d_ref[...]) * 0.25


def kernel(edge_index, edge_weight, user_emb, item_emb):
    ei = edge_index.astype(jnp.int32)
    w = edge_weight.astype(jnp.float32)

    n_edges = w.shape[0]
    per_tile = NUM_TILES * B * SB_ROWS
    e_pad = ((n_edges + per_tile - 1) // per_tile) * per_tile
    pad = e_pad - n_edges
    if pad:
        ei = jnp.concatenate([ei, jnp.zeros((2, pad), jnp.int32)], axis=1)
        w = jnp.concatenate([w, jnp.zeros((pad,), jnp.float32)])
    ei3 = ei.reshape(2, -1, B)
    w2 = w.reshape(-1, B)

    ego = jnp.concatenate(
        [user_emb, item_emb, jnp.zeros((N_PAD - N, 2 * H), jnp.float32)],
        axis=0)
    e0b = jnp.concatenate([ego[:, :H], ego[:, H:]], axis=0)
    zeros = jnp.zeros((CH, H), jnp.float32)

    mesh = plsc.VectorSubcoreMesh(core_axis_name="c", subcore_axis_name="s")
    out_t = [
        jax.ShapeDtypeStruct((2 * N_PAD, H), jnp.float32),
        jax.ShapeDtypeStruct((2 * N_PAD, H), jnp.float32),
        jax.ShapeDtypeStruct((N_PAD, 2 * H), jnp.float32),
    ]
    _, _, mout = pl.kernel(
        _sc_body,
        out_type=out_t,
        mesh=mesh,
        compiler_params=pltpu.CompilerParams(use_tc_tiling_on_sc=False),
        scratch_types=[
            pltpu.VMEM((2, SB_ROWS, B), jnp.int32),
            pltpu.VMEM((2, SB_ROWS, B), jnp.int32),
            pltpu.VMEM((2, SB_ROWS, B), jnp.float32),
            pltpu.VMEM((3, B, H), jnp.float32),
            pltpu.VMEM((CH, H), jnp.float32),
            pltpu.VMEM((CH, H), jnp.float32),
            pltpu.VMEM((CH, H), jnp.float32),
            pltpu.VMEM((CH, H), jnp.float32),
            pltpu.VMEM((CH, H), jnp.float32),
            pltpu.VMEM_SHARED((N_PAD, H), jnp.float32),
            pltpu.SemaphoreType.DMA((3,)),
            pltpu.SemaphoreType.DMA((3,)),
            pltpu.SemaphoreType.DMA((2,)),
            pltpu.SemaphoreType.DMA,
            pltpu.SemaphoreType.DMA,
        ],
    )(e0b, ei3, w2, zeros)

    return mout[:N_USERS], mout[N_USERS:N]
